# baseline (speedup 1.0000x reference)
import jax
import jax.numpy as jnp
from jax.experimental import pallas as pl

_EPS = 1e-12
_BLOCK_ROWS = 8000


def _normalize_block(x_ref, o_ref):
    x = x_ref[...]
    s = jnp.sum(x * x, axis=1, keepdims=True)
    n = jnp.maximum(jnp.sqrt(s), _EPS)
    o_ref[...] = x / n


def kernel(weight):
    n_rows, dim = weight.shape
    grid = n_rows // _BLOCK_ROWS
    return pl.pallas_call(
        _normalize_block,
        grid=(grid,),
        in_specs=[pl.BlockSpec((_BLOCK_ROWS, dim), lambda i: (i, 0))],
        out_specs=pl.BlockSpec((_BLOCK_ROWS, dim), lambda i: (i, 0)),
        out_shape=jax.ShapeDtypeStruct((n_rows, dim), weight.dtype),
    )(weight)
